# Initial kernel scaffold; baseline (speedup 1.0000x reference)
import jax
import jax.numpy as jnp
from jax.experimental import pallas as pl


def kernel(f_BChw, emb, conv_w, conv_b):
    raise NotImplementedError("write your pallas kernel here")


# speedup vs baseline: 1.3805x; 1.3805x over previous
import functools

import jax
import jax.numpy as jnp
from jax import lax
from jax.experimental import pallas as pl
from jax.experimental.pallas import tpu as pltpu
from jax.experimental.pallas import tpu_sc as plsc

B, C, H, W = 32, 64, 32, 32
HW = H * W
N = B * HW
V = 8192
RB = 256
QB = HW // RB
NB = N // RB
BETA = 0.25
RESI = 0.5

NWORK = 32
IDX_ROWS = N // 128
ROWS_PER_W = IDX_ROWS // NWORK
VSLICE = V // NWORK


def _dist_body(f_ref, emb_ref, idx_ref, en_ref):
    b = pl.program_id(0)
    q = pl.program_id(1)

    @pl.when((b == 0) & (q == 0))
    def _():
        e = emb_ref[...]
        ones_row = jnp.ones((1, C), jnp.float32)
        en_ref[...] = lax.dot_general(
            ones_row, e * e, (((1,), (1,)), ((), ())),
            precision=lax.Precision.HIGHEST,
            preferred_element_type=jnp.float32)

    fbt = f_ref[0].T
    m = lax.dot_general(fbt, emb_ref[...], (((1,), (1,)), ((), ())),
                        preferred_element_type=jnp.float32)
    xn = jnp.sum(fbt * fbt, axis=1, keepdims=True)
    d = (xn + en_ref[...]) - 2.0 * m
    idx = jnp.argmin(d, axis=1).astype(jnp.int32)
    idx_ref[...] = idx.reshape(1, 1, RB)


_dist = pl.pallas_call(
    _dist_body,
    grid=(B, QB),
    in_specs=[
        pl.BlockSpec((1, C, RB), lambda b, q: (b, 0, q)),
        pl.BlockSpec((V, C), lambda b, q: (0, 0)),
    ],
    out_specs=pl.BlockSpec((1, 1, RB), lambda b, q: (b * QB + q, 0, 0)),
    out_shape=jax.ShapeDtypeStruct((NB, 1, RB), jnp.int32),
    scratch_shapes=[pltpu.VMEM((1, V), jnp.float32)],
)


def _sc_body(idx_hbm, emb_hbm, h_out, hit_out, idxb, rows, hist, sem):
    wid = lax.axis_index("s") * 2 + lax.axis_index("c")
    base = wid * ROWS_PER_W

    pltpu.sync_copy(idx_hbm.at[pl.ds(base, ROWS_PER_W)], idxb)
    half = ROWS_PER_W // 2
    for r in range(2):
        cps = [pltpu.async_copy(emb_hbm.at[idxb.at[r * half + j]],
                                rows.at[j], sem)
               for j in range(half)]
        for cp in cps:
            cp.wait()
        pltpu.sync_copy(rows, h_out.at[pl.ds(base + r * half, half)])

    lo = wid * VSLICE
    zeros16 = jnp.zeros((16,), jnp.float32)
    ones16 = jnp.ones((16,), jnp.float32)
    for k in range(VSLICE // 16):
        hist[pl.ds(k * 16, 16)] = zeros16

    def chunk_body(c, carry):
        pltpu.sync_copy(idx_hbm.at[pl.ds(c * ROWS_PER_W, ROWS_PER_W)], idxb)
        for j in range(ROWS_PER_W):
            for k in range(128 // 16):
                v = idxb[j, pl.ds(k * 16, 16)]
                m = (v >= lo) & (v < lo + VSLICE)
                plsc.addupdate_scatter(hist, [v - lo], ones16, mask=m)
        return carry

    lax.fori_loop(0, IDX_ROWS // ROWS_PER_W, chunk_body, 0)
    pltpu.sync_copy(hist, hit_out.at[pl.ds(lo, VSLICE)])


@functools.cache
def _sc_gather():
    return pl.kernel(
        _sc_body,
        mesh=plsc.VectorSubcoreMesh(core_axis_name="c", subcore_axis_name="s"),
        out_type=[
            jax.ShapeDtypeStruct((IDX_ROWS, 128, 128), jnp.float32),
            jax.ShapeDtypeStruct((V,), jnp.float32),
        ],
        scratch_types=[
            pltpu.VMEM((ROWS_PER_W, 128), jnp.int32),
            pltpu.VMEM((ROWS_PER_W // 2, 128, 128), jnp.float32),
            pltpu.VMEM((VSLICE,), jnp.float32),
            pltpu.SemaphoreType.DMA,
        ],
        compiler_params=pltpu.CompilerParams(needs_layout_passes=False),
    )


def _conv_body(h_ref, f_ref, wt_ref, b_ref, out_ref, loss_ref):
    bi = pl.program_id(0)
    x = h_ref[0, :, :C]
    wcol = lax.broadcasted_iota(jnp.int32, (HW, C), 0) % W
    acc = jnp.zeros((HW, C), jnp.float32)
    for ky in range(3):
        dy = ky - 1
        for kx in range(3):
            dx = kx - 1
            s = dy * W + dx
            if s > 0:
                patch = jnp.concatenate(
                    [x[s:], jnp.zeros((s, C), jnp.float32)], axis=0)
            elif s < 0:
                patch = jnp.concatenate(
                    [jnp.zeros((-s, C), jnp.float32), x[:HW + s]], axis=0)
            else:
                patch = x
            if dx == 1:
                patch = jnp.where(wcol == W - 1, 0.0, patch)
            elif dx == -1:
                patch = jnp.where(wcol == 0, 0.0, patch)
            acc = acc + jnp.dot(patch, wt_ref[ky, kx],
                                preferred_element_type=jnp.float32)
    fh = x * (1.0 - RESI) + (acc + b_ref[...]) * RESI
    fh_t = fh.T
    out_ref[0] = fh_t
    dlt = fh_t - f_ref[0]
    part = jnp.sum(dlt * dlt).reshape(1, 1)

    @pl.when(bi == 0)
    def _():
        loss_ref[...] = jnp.zeros((1, 1), jnp.float32)

    loss_ref[...] += part

    @pl.when(bi == B - 1)
    def _():
        loss_ref[...] = loss_ref[...] * ((1.0 + BETA) / (B * C * HW))


_conv = pl.pallas_call(
    _conv_body,
    grid=(B,),
    in_specs=[
        pl.BlockSpec((1, HW, 128), lambda b: (b, 0, 0)),
        pl.BlockSpec((1, C, HW), lambda b: (b, 0, 0)),
        pl.BlockSpec((3, 3, C, C), lambda b: (0, 0, 0, 0)),
        pl.BlockSpec((1, C), lambda b: (0, 0)),
    ],
    out_specs=[
        pl.BlockSpec((1, C, HW), lambda b: (b, 0, 0)),
        pl.BlockSpec((1, 1), lambda b: (0, 0)),
    ],
    out_shape=[
        jax.ShapeDtypeStruct((B, C, HW), jnp.float32),
        jax.ShapeDtypeStruct((1, 1), jnp.float32),
    ],
)


def kernel(f_BChw, emb, conv_w, conv_b):
    f3 = f_BChw.reshape(B, C, HW)
    idx_blocks = _dist(f3, emb)
    idx2 = idx_blocks.reshape(IDX_ROWS, 128)
    emb_pad = jnp.pad(emb, ((0, 0), (0, 128 - C)))
    h3, hit_V = _sc_gather()(idx2, emb_pad)
    h_b = h3.reshape(B, HW, 128)
    wt = jnp.transpose(conv_w, (2, 3, 1, 0))
    f_hat3, loss11 = _conv(h_b, f3, wt, conv_b.reshape(1, C))
    return (f_hat3.reshape(B, C, H, W), loss11[0, 0], hit_V)


# speedup vs baseline: 1.4634x; 1.0600x over previous
import functools

import jax
import jax.numpy as jnp
from jax import lax
from jax.experimental import pallas as pl
from jax.experimental.pallas import tpu as pltpu
from jax.experimental.pallas import tpu_sc as plsc

B, C, H, W = 32, 64, 32, 32
HW = H * W
N = B * HW
V = 8192
RB = 256
QB = HW // RB
NB = N // RB
BETA = 0.25
RESI = 0.5

NWORK = 32
IDX_ROWS = N // 128
ROWS_PER_W = IDX_ROWS // NWORK
VSLICE = V // NWORK


def _dist_body(f_ref, emb_ref, idx_ref, en_ref):
    b = pl.program_id(0)
    q = pl.program_id(1)

    @pl.when((b == 0) & (q == 0))
    def _():
        e = emb_ref[...]
        ones_row = jnp.ones((1, C), jnp.float32)
        en_ref[...] = lax.dot_general(
            ones_row, e * e, (((1,), (1,)), ((), ())),
            precision=lax.Precision.HIGHEST,
            preferred_element_type=jnp.float32)

    fbt = f_ref[0].T
    m = lax.dot_general(fbt, emb_ref[...], (((1,), (1,)), ((), ())),
                        preferred_element_type=jnp.float32)
    xn = jnp.sum(fbt * fbt, axis=1, keepdims=True)
    d = (xn + en_ref[...]) - 2.0 * m
    idx = jnp.argmin(d, axis=1).astype(jnp.int32)
    idx_ref[...] = idx.reshape(1, 1, RB)


_dist = pl.pallas_call(
    _dist_body,
    grid=(B, QB),
    in_specs=[
        pl.BlockSpec((1, C, RB), lambda b, q: (b, 0, q)),
        pl.BlockSpec((V, C), lambda b, q: (0, 0)),
    ],
    out_specs=pl.BlockSpec((1, 1, RB), lambda b, q: (b * QB + q, 0, 0)),
    out_shape=jax.ShapeDtypeStruct((NB, 1, RB), jnp.int32),
    scratch_shapes=[pltpu.VMEM((1, V), jnp.float32)],
)


def _sc_body(idx_hbm, emb_hbm, h_out, hit_out, idx_all, rows, hist, sem):
    wid = lax.axis_index("s") * 2 + lax.axis_index("c")
    base = wid * ROWS_PER_W
    half = ROWS_PER_W // 2

    pltpu.sync_copy(idx_hbm, idx_all)

    lo = wid * VSLICE
    zeros16 = jnp.zeros((16,), jnp.float32)
    ones16 = jnp.ones((16,), jnp.float32)
    for k in range(VSLICE // 16):
        hist[pl.ds(k * 16, 16)] = zeros16

    def scan_rows(r0, rn):
        def row_body(r, carry):
            for k in range(128 // 16):
                v = idx_all[r, pl.ds(k * 16, 16)]
                m = (v >= lo) & (v < lo + VSLICE)
                plsc.addupdate_scatter(hist, [v - lo], ones16, mask=m)
            return carry
        lax.fori_loop(r0, r0 + rn, row_body, 0)

    for r in range(2):
        cps = [pltpu.async_copy(emb_hbm.at[idx_all.at[base + r * half + j]],
                                rows.at[j], sem)
               for j in range(half)]
        scan_rows(r * (IDX_ROWS // 2), IDX_ROWS // 2)
        for cp in cps:
            cp.wait()
        pltpu.sync_copy(rows, h_out.at[pl.ds(base + r * half, half)])

    pltpu.sync_copy(hist, hit_out.at[pl.ds(lo, VSLICE)])


@functools.cache
def _sc_gather():
    return pl.kernel(
        _sc_body,
        mesh=plsc.VectorSubcoreMesh(core_axis_name="c", subcore_axis_name="s"),
        out_type=[
            jax.ShapeDtypeStruct((IDX_ROWS, 128, 128), jnp.float32),
            jax.ShapeDtypeStruct((V,), jnp.float32),
        ],
        scratch_types=[
            pltpu.VMEM((IDX_ROWS, 128), jnp.int32),
            pltpu.VMEM((ROWS_PER_W // 2, 128, 128), jnp.float32),
            pltpu.VMEM((VSLICE,), jnp.float32),
            pltpu.SemaphoreType.DMA,
        ],
        compiler_params=pltpu.CompilerParams(needs_layout_passes=False),
    )


def _conv_body(h_ref, f_ref, wt_ref, b_ref, out_ref, loss_ref):
    bi = pl.program_id(0)
    x = h_ref[0, :, :C]
    wcol = lax.broadcasted_iota(jnp.int32, (HW, C), 0) % W
    acc = jnp.zeros((HW, C), jnp.float32)
    for ky in range(3):
        dy = ky - 1
        for kx in range(3):
            dx = kx - 1
            s = dy * W + dx
            if s > 0:
                patch = jnp.concatenate(
                    [x[s:], jnp.zeros((s, C), jnp.float32)], axis=0)
            elif s < 0:
                patch = jnp.concatenate(
                    [jnp.zeros((-s, C), jnp.float32), x[:HW + s]], axis=0)
            else:
                patch = x
            if dx == 1:
                patch = jnp.where(wcol == W - 1, 0.0, patch)
            elif dx == -1:
                patch = jnp.where(wcol == 0, 0.0, patch)
            acc = acc + jnp.dot(patch, wt_ref[ky, kx],
                                preferred_element_type=jnp.float32)
    fh = x * (1.0 - RESI) + (acc + b_ref[...]) * RESI
    fh_t = fh.T
    out_ref[0] = fh_t
    dlt = fh_t - f_ref[0]
    part = jnp.sum(dlt * dlt).reshape(1, 1)

    @pl.when(bi == 0)
    def _():
        loss_ref[...] = jnp.zeros((1, 1), jnp.float32)

    loss_ref[...] += part

    @pl.when(bi == B - 1)
    def _():
        loss_ref[...] = loss_ref[...] * ((1.0 + BETA) / (B * C * HW))


_conv = pl.pallas_call(
    _conv_body,
    grid=(B,),
    in_specs=[
        pl.BlockSpec((1, HW, 128), lambda b: (b, 0, 0)),
        pl.BlockSpec((1, C, HW), lambda b: (b, 0, 0)),
        pl.BlockSpec((3, 3, C, C), lambda b: (0, 0, 0, 0)),
        pl.BlockSpec((1, C), lambda b: (0, 0)),
    ],
    out_specs=[
        pl.BlockSpec((1, C, HW), lambda b: (b, 0, 0)),
        pl.BlockSpec((1, 1), lambda b: (0, 0)),
    ],
    out_shape=[
        jax.ShapeDtypeStruct((B, C, HW), jnp.float32),
        jax.ShapeDtypeStruct((1, 1), jnp.float32),
    ],
)


def kernel(f_BChw, emb, conv_w, conv_b):
    f3 = f_BChw.reshape(B, C, HW)
    idx_blocks = _dist(f3, emb)
    idx2 = idx_blocks.reshape(IDX_ROWS, 128)
    emb_pad = jnp.pad(emb, ((0, 0), (0, 128 - C)))
    h3, hit_V = _sc_gather()(idx2, emb_pad)
    h_b = h3.reshape(B, HW, 128)
    wt = jnp.transpose(conv_w, (2, 3, 1, 0))
    f_hat3, loss11 = _conv(h_b, f3, wt, conv_b.reshape(1, C))
    return (f_hat3.reshape(B, C, H, W), loss11[0, 0], hit_V)
